# baseline (speedup 1.0000x reference)
import functools

import jax
import jax.numpy as jnp
from jax import lax
from jax.experimental import pallas as pl
from jax.experimental.pallas import tpu as pltpu
from jax.experimental.pallas import tpu_sc as plsc

EMB = 50
WPAD = 64
NC, NS = 2, 16
NW = NC * NS
IDX_TILE = 128
TILES_PER_CHUNK = 8
CHUNK = IDX_TILE * TILES_PER_CHUNK


@functools.cache
def _make(B):
    assert B % (NW * CHUNK) == 0
    b_per_w = B // NW
    n_chunks = b_per_w // CHUNK
    mesh = plsc.VectorSubcoreMesh(core_axis_name="c", subcore_axis_name="s")

    @functools.partial(
        pl.kernel,
        mesh=mesh,
        out_type=jax.ShapeDtypeStruct((B, EMB), jnp.float32),
        compiler_params=pltpu.CompilerParams(use_tc_tiling_on_sc=False),
        scratch_types=[
            pltpu.VMEM((TILES_PER_CHUNK, IDX_TILE), jnp.int32),
            pltpu.VMEM((CHUNK, WPAD), jnp.float32),
            pltpu.VMEM((CHUNK, EMB), jnp.float32),
            pltpu.VMEM_SHARED((128, WPAD), jnp.float32),
            pltpu.SemaphoreType.DMA,
        ],
    )
    def k(batch_hbm, table_hbm, out_hbm, idx_v, rows_v, dense_v, table_sh, sem):
        s = lax.axis_index("s")
        wid = s * NC + lax.axis_index("c")

        @pl.when(s == 0)
        def _():
            pltpu.sync_copy(table_hbm, table_sh)

        plsc.subcore_barrier()

        def step(i, carry):
            base = wid * b_per_w + i * CHUNK
            rowbase = base // IDX_TILE
            pltpu.sync_copy(batch_hbm.at[pl.ds(rowbase, TILES_PER_CHUNK)], idx_v)
            cps = []
            for j in range(TILES_PER_CHUNK):
                cp = pltpu.make_async_copy(
                    table_hbm.at[idx_v.at[j]],
                    rows_v.at[pl.ds(j * IDX_TILE, IDX_TILE)],
                    sem,
                )
                cp.start()
                cps.append(cp)
            for cp in cps:
                cp.wait()

            @plsc.parallel_loop(0, CHUNK, unroll=4)
            def _row(r):
                for off in (0, 16, 32, 34):
                    dense_v[r, pl.ds(off, 16)] = rows_v[r, pl.ds(off, 16)]

            pltpu.sync_copy(dense_v, out_hbm.at[pl.ds(base, CHUNK)])
            return carry

        lax.fori_loop(0, n_chunks, step, 0)

    return k


def kernel(batch, table):
    R, C = batch.shape
    B = R * C
    flat = batch.reshape(B // IDX_TILE, IDX_TILE).astype(jnp.int32)
    tpad = jnp.zeros((table.shape[0], WPAD), jnp.float32).at[:, :EMB].set(table)
    out = _make(B)(flat, tpad)
    return out.reshape(R, C, EMB)


# speedup vs baseline: 1.6714x; 1.6714x over previous
import functools

import jax
import jax.numpy as jnp
from jax import lax
from jax.experimental import pallas as pl
from jax.experimental.pallas import tpu as pltpu
from jax.experimental.pallas import tpu_sc as plsc

EMB = 50
WPAD = 64
NC, NS = 2, 16
NW = NC * NS
IDX_TILE = 128
TILES_PER_CHUNK = 4
CHUNK = IDX_TILE * TILES_PER_CHUNK
NBUF = 2


@functools.cache
def _make(B):
    assert B % (NW * CHUNK * NBUF) == 0
    b_per_w = B // NW
    n_chunks = b_per_w // CHUNK
    mesh = plsc.VectorSubcoreMesh(core_axis_name="c", subcore_axis_name="s")

    @functools.partial(
        pl.kernel,
        mesh=mesh,
        out_type=jax.ShapeDtypeStruct((B, EMB), jnp.float32),
        compiler_params=pltpu.CompilerParams(use_tc_tiling_on_sc=False),
        scratch_types=[
            pltpu.VMEM((NBUF, TILES_PER_CHUNK, IDX_TILE), jnp.int32),
            pltpu.VMEM((NBUF, CHUNK, WPAD), jnp.float32),
            pltpu.VMEM((NBUF, CHUNK, EMB), jnp.float32),
            pltpu.VMEM_SHARED((128, WPAD), jnp.float32),
            pltpu.SemaphoreType.DMA,
            pltpu.SemaphoreType.DMA,
        ],
    )
    def k(batch_hbm, table_hbm, out_hbm, idx_v, rows_v, dense_v, table_sh,
          sem_g, sem_o):
        s = lax.axis_index("s")
        wid = s * NC + lax.axis_index("c")
        w0 = wid * b_per_w

        @pl.when(s == 0)
        def _():
            pltpu.sync_copy(table_hbm, table_sh)

        plsc.subcore_barrier()

        def fire_gathers(i, b):
            rowbase = (w0 + i * CHUNK) // IDX_TILE
            pltpu.sync_copy(
                batch_hbm.at[pl.ds(rowbase, TILES_PER_CHUNK)], idx_v.at[b])
            for j in range(TILES_PER_CHUNK):
                pltpu.make_async_copy(
                    table_sh.at[idx_v.at[b].at[j]],
                    rows_v.at[b].at[pl.ds(j * IDX_TILE, IDX_TILE)],
                    sem_g,
                ).start()

        def wait_gathers(b):
            for j in range(TILES_PER_CHUNK):
                pltpu.make_async_copy(
                    table_sh.at[idx_v.at[b].at[j]],
                    rows_v.at[b].at[pl.ds(j * IDX_TILE, IDX_TILE)],
                    sem_g,
                ).wait()

        def out_copy(i, b):
            return pltpu.make_async_copy(
                dense_v.at[b], out_hbm.at[pl.ds(w0 + i * CHUNK, CHUNK)], sem_o)

        def process(i, b):
            wait_gathers(b)

            @plsc.parallel_loop(0, CHUNK, unroll=4)
            def _row(r):
                for off in (0, 16, 32, 34):
                    dense_v[b, r, pl.ds(off, 16)] = rows_v[b, r, pl.ds(off, 16)]

            out_copy(i, b).start()

        fire_gathers(0, 0)

        def step(t, carry):
            i0 = t * 2

            fire_gathers(i0 + 1, 1)

            @pl.when(t > 0)
            def _():
                out_copy(i0 - 2, 0).wait()

            process(i0, 0)

            @pl.when(i0 + 2 < n_chunks)
            def _():
                fire_gathers(i0 + 2, 0)

            @pl.when(t > 0)
            def _():
                out_copy(i0 - 1, 1).wait()

            process(i0 + 1, 1)
            return carry

        lax.fori_loop(0, n_chunks // 2, step, 0)
        out_copy(n_chunks - 2, 0).wait()
        out_copy(n_chunks - 1, 1).wait()

    return k


def kernel(batch, table):
    R, C = batch.shape
    B = R * C
    flat = batch.reshape(B // IDX_TILE, IDX_TILE).astype(jnp.int32)
    tpad = jnp.zeros((table.shape[0], WPAD), jnp.float32).at[:, :EMB].set(table)
    out = _make(B)(flat, tpad)
    return out.reshape(R, C, EMB)
